# Initial kernel scaffold; baseline (speedup 1.0000x reference)
import jax
import jax.numpy as jnp
from jax.experimental import pallas as pl


def kernel(x, x_knn, knn_idx, p_r, W, b, Wx, bx, Wp1, gamma, beta, Wp2, bp2):
    raise NotImplementedError("write your pallas kernel here")


# speedup vs baseline: 4.6280x; 4.6280x over previous
import functools

import jax
import jax.numpy as jnp
from jax import lax
from jax.experimental import pallas as pl
from jax.experimental.pallas import tpu as pltpu
from jax.experimental.pallas import tpu_sc as plsc

_NC, _NS, _L = 2, 16, 16
_NW = _NC * _NS
_CHUNK = 128


def _stats_body(p_ref, q_ref):
    p = p_ref[...]
    q_ref[...] = lax.dot_general(
        p, p, (((0,), (0,)), ((), ())), preferred_element_type=jnp.float32)


def _make_main_body(share):
    def _main_body(x_ref, p_ref, w_ref, wx_ref, a_ref, c_ref, b2_ref,
                   bias_ref, bx_ref, sv_ref, gmax_ref):
        x = x_ref[...]
        s = jnp.dot(x, w_ref[...], preferred_element_type=jnp.float32)
        hr = jnp.maximum(
            jnp.dot(p_ref[...], a_ref[...],
                    preferred_element_type=jnp.float32) + c_ref[...], 0.0)
        s = s + jnp.dot(hr, b2_ref[...],
                        preferred_element_type=jnp.float32) + bias_ref[...]
        v = jnp.dot(x, wx_ref[...], preferred_element_type=jnp.float32) + bx_ref[...]
        sv_ref[...] = jnp.concatenate(
            [s, v, jnp.zeros((s.shape[0], 4 * _L), s.dtype)], axis=1)

        @pl.when(pl.program_id(0) == 0)
        def _():
            gmax_ref[0, 0] = -jnp.inf

        gmax_ref[0, 0] = jnp.maximum(gmax_ref[0, 0], jnp.max(s))
    return _main_body


def _make_scatter(m, n_pad):
    nchunks = m // _CHUNK
    iters = (nchunks + _NW - 1) // _NW
    rz = n_pad // _NS
    mesh = plsc.VectorSubcoreMesh(core_axis_name="c", subcore_axis_name="s")

    @functools.partial(
        pl.kernel,
        out_type=jax.ShapeDtypeStruct((_NC * n_pad, 8 * _L), jnp.float32),
        mesh=mesh,
        scratch_types=[
            pltpu.VMEM((_CHUNK,), jnp.int32),
            pltpu.VMEM((_CHUNK, 8 * _L), jnp.float32),
            pltpu.VMEM((_L,), jnp.float32),
            pltpu.VMEM((8, 8 * _L), jnp.float32),
            pltpu.VMEM_SHARED((n_pad, 8 * _L), jnp.float32),
        ],
    )
    def _scatter(sv_hbm, idx_hbm, g_hbm, out_hbm, idxb, svb, gb, zb, acc):
        cid = lax.axis_index("c")
        sid = lax.axis_index("s")
        wid = sid * _NC + cid

        def zrow(r, carry):
            for j in range(8):
                zb[r, pl.ds(j * _L, _L)] = jnp.zeros((_L,), jnp.float32)
            return carry

        lax.fori_loop(0, 8, zrow, 0)

        def zslab(r, carry):
            pltpu.sync_copy(zb, acc.at[pl.ds(sid * rz + r * 8, 8)])
            return carry

        lax.fori_loop(0, rz // 8, zslab, 0)
        pltpu.sync_copy(g_hbm, gb)
        gv = gb[...]
        plsc.subcore_barrier()

        def body(j, carry):
            chunk = wid + _NW * j

            @pl.when(chunk < nchunks)
            def _():
                base = chunk * _CHUNK
                pltpu.sync_copy(idx_hbm.at[pl.ds(base, _CHUNK)], idxb)
                pltpu.sync_copy(sv_hbm.at[pl.ds(base, _CHUNK)], svb)

                def rbody(r, c2):
                    s0 = svb[r, pl.ds(0, _L)]
                    s1 = svb[r, pl.ds(_L, _L)]
                    v0 = svb[r, pl.ds(2 * _L, _L)]
                    v1 = svb[r, pl.ds(3 * _L, _L)]
                    e0 = jnp.exp(s0 - gv)
                    e1 = jnp.exp(s1 - gv)
                    svb[r, pl.ds(0, _L)] = e0
                    svb[r, pl.ds(_L, _L)] = e1
                    svb[r, pl.ds(2 * _L, _L)] = v0 * e0
                    svb[r, pl.ds(3 * _L, _L)] = v1 * e1
                    return c2

                lax.fori_loop(0, _CHUNK, rbody, 0)
                pltpu.sync_copy(svb, acc.at[idxb], add=True)

            return carry

        lax.fori_loop(0, iters, body, 0)
        plsc.subcore_barrier()
        pltpu.sync_copy(acc.at[pl.ds(sid * rz, rz)],
                        out_hbm.at[pl.ds(cid * n_pad + sid * rz, rz)])

    return _scatter


def _make_combine_body(hid, share):
    def _combine_body(p0_ref, p1_ref, x_ref, out_ref):
        den = p0_ref[:, 0:hid] + p1_ref[:, 0:hid]
        num = p0_ref[:, hid:2 * hid] + p1_ref[:, hid:2 * hid]
        res = jnp.where(den > 0.0, num / den, 0.0)
        out_ref[...] = x_ref[...] + jnp.concatenate([res] * share, axis=1)
    return _combine_body


def kernel(x, x_knn, knn_idx, p_r, W, b, Wx, bx, Wp1, gamma, beta, Wp2, bp2):
    n, k, c = x_knn.shape
    hid = W.shape[1]
    share = c // hid
    m = n * k
    f32 = jnp.float32

    xe = x_knn.reshape(m, c)
    p49 = jnp.concatenate(
        [p_r.reshape(n, k * 3), jnp.ones((n, 1), f32)], axis=1)

    q = pl.pallas_call(
        _stats_body,
        out_shape=jax.ShapeDtypeStruct((k * 3 + 1, k * 3 + 1), f32),
    )(p49)

    q48 = q[:k * 3, :k * 3].reshape(k, 3, k, 3)
    c3 = jnp.einsum('iaib->ab', q48)
    s3 = q[k * 3, :k * 3].reshape(k, 3).sum(axis=0)
    mean = (s3 / m) @ Wp1
    eh2 = jnp.einsum('ij,ik,kj->j', Wp1, c3 / m, Wp1)
    var = eh2 - mean * mean
    a = gamma * lax.rsqrt(var + 1e-5)
    cshift = beta - mean * a
    a4 = jnp.zeros((4, 4), f32).at[:3, :3].set(Wp1 * a[None, :])
    c4 = jnp.zeros((1, 4), f32).at[0, :3].set(cshift)
    b24 = jnp.zeros((4, hid), f32).at[:3, :].set(Wp2 @ W)
    b2 = (b + bp2 @ W).reshape(1, hid)
    bx2 = bx.reshape(1, hid)
    p4 = jnp.pad(p_r.reshape(m, 3), ((0, 0), (0, 1)))

    be = 2000
    sv, gmax = pl.pallas_call(
        _make_main_body(share),
        grid=(m // be,),
        in_specs=[
            pl.BlockSpec((be, c), lambda i: (i, 0)),
            pl.BlockSpec((be, 4), lambda i: (i, 0)),
            pl.BlockSpec((c, hid), lambda i: (0, 0)),
            pl.BlockSpec((c, hid), lambda i: (0, 0)),
            pl.BlockSpec((4, 4), lambda i: (0, 0)),
            pl.BlockSpec((1, 4), lambda i: (0, 0)),
            pl.BlockSpec((4, hid), lambda i: (0, 0)),
            pl.BlockSpec((1, hid), lambda i: (0, 0)),
            pl.BlockSpec((1, hid), lambda i: (0, 0)),
        ],
        out_specs=[
            pl.BlockSpec((be, 8 * _L), lambda i: (i, 0)),
            pl.BlockSpec((1, 1), lambda i: (0, 0), memory_space=pltpu.SMEM),
        ],
        out_shape=[
            jax.ShapeDtypeStruct((m, 8 * _L), f32),
            jax.ShapeDtypeStruct((1, 1), f32),
        ],
    )(xe, p4, W, Wx, a4, c4, b24, b2, bx2)

    gvec = jnp.full((_L,), gmax[0, 0], f32)

    n_pad = ((n + _NS * 8 - 1) // (_NS * 8)) * (_NS * 8)
    parts = _make_scatter(m, n_pad)(sv, knn_idx.reshape(m), gvec)
    p0 = lax.slice(parts, (0, 0), (n, 2 * hid))
    p1 = lax.slice(parts, (n_pad, 0), (n_pad + n, 2 * hid))

    bn = 2000
    nb = n // bn
    out = pl.pallas_call(
        _make_combine_body(hid, share),
        grid=(nb,),
        in_specs=[
            pl.BlockSpec((bn, 2 * hid), lambda i: (i, 0)),
            pl.BlockSpec((bn, 2 * hid), lambda i: (i, 0)),
            pl.BlockSpec((bn, c), lambda i: (i, 0)),
        ],
        out_specs=pl.BlockSpec((bn, c), lambda i: (i, 0)),
        out_shape=jax.ShapeDtypeStruct((n, c), f32),
    )(p0, p1, x)
    return out
